# Initial kernel scaffold; baseline (speedup 1.0000x reference)
import jax
import jax.numpy as jnp
from jax.experimental import pallas as pl


def kernel(positions, sizes, densities, colors, camera_matrix, intrinsics):
    raise NotImplementedError("write your pallas kernel here")


# speedup vs baseline: 26.4069x; 26.4069x over previous
import functools

import jax
import jax.numpy as jnp
from jax.experimental import pallas as pl
from jax.experimental.pallas import tpu as pltpu

_H = 512
_W = 512
_SIGMA = 1.0
_TEMP = 0.1
_LANES = 128


def _proj_body(sx_ref, sy_ref, dep_ref_in, ss_ref, sizes_ref, dens_ref,
               pa_ref, pid_ref):
    sx = sx_ref[...]
    sy = sy_ref[...]
    depth = dep_ref_in[...]
    ss = ss_ref[...]
    valid = ((depth > 0.1) & (depth < 100.0)
             & (sx + ss >= 0) & (sx - ss < _W)
             & (sy + ss >= 0) & (sy - ss < _H))
    pxf = jnp.clip(jnp.round(sx), 0.0, _W - 1.0)
    pyf = jnp.clip(jnp.round(sy), 0.0, _H - 1.0)
    dx = pxf - sx
    dy = pyf - sy
    dist = jnp.sqrt(dx * dx + dy * dy + 1e-12)
    inside = (dist <= jnp.maximum(ss * 0.5, 0.71)).astype(jnp.float32)
    w = jnp.clip(jnp.exp(-dist / (2.0 * _SIGMA ** 2)), 0.0, 1.0)
    dens = dens_ref[...]
    sig = jnp.maximum(dens, 0.0) + jnp.log1p(jnp.exp(-jnp.abs(dens)))
    v_alpha = jnp.clip(1.0 - jnp.exp(-sig * sizes_ref[...]), 0.0, 1.0)
    pa = v_alpha * w * inside * valid.astype(jnp.float32)
    px = jnp.where(valid, pxf, 0.0).astype(jnp.int32)
    py = jnp.where(valid, pyf, 0.0).astype(jnp.int32)
    pa_ref[...] = pa
    pid_ref[...] = py * _W + px


def _project(sx2, sy2, dep2, ss2, sizes2, dens2, interpret=False):
    rows = sizes2.shape[0]
    shape = (rows, _LANES)
    return pl.pallas_call(
        _proj_body,
        in_specs=[pl.BlockSpec(memory_space=pltpu.VMEM)] * 6,
        out_specs=[pl.BlockSpec(memory_space=pltpu.VMEM)] * 2,
        out_shape=[
            jax.ShapeDtypeStruct(shape, jnp.float32),
            jax.ShapeDtypeStruct(shape, jnp.int32),
        ],
        interpret=interpret,
    )(sx2, sy2, dep2, ss2, sizes2, dens2)


def _blend_body(n_steps, pidf_ref, pa_ref, dep_ref, r_ref, g_ref, b_ref,
                R, G, B, D, A):
    nrows = _H * _W // _LANES
    R[...] = jnp.zeros((nrows, _LANES), jnp.float32)
    G[...] = jnp.zeros((nrows, _LANES), jnp.float32)
    B[...] = jnp.zeros((nrows, _LANES), jnp.float32)
    A[...] = jnp.zeros((nrows, _LANES), jnp.float32)
    D[...] = jnp.full((nrows, _LANES), 100.0, jnp.float32)

    iota = jax.lax.broadcasted_iota(jnp.int32, (1, _LANES), 1)

    def step(i, _):
        rr = i // _LANES
        cc = i - rr * _LANES
        frow = pl.ds(rr, 1)
        cmask = (iota == cc).astype(jnp.float32)
        pa = jnp.sum(pa_ref[frow, :] * cmask)
        dep = jnp.sum(dep_ref[frow, :] * cmask)
        cr = jnp.sum(r_ref[frow, :] * cmask)
        cg = jnp.sum(g_ref[frow, :] * cmask)
        cb = jnp.sum(b_ref[frow, :] * cmask)
        pf = jnp.sum(pidf_ref[frow, :] * cmask)
        srf = jnp.floor(pf * (1.0 / _LANES))
        lxf = pf - srf * _LANES
        sr = srf.astype(jnp.int32)
        lx = lxf.astype(jnp.int32)
        ys = pl.ds(sr, 1)
        pmask = iota == lx
        cura_row = A[ys, :]
        blend_row = jnp.clip(pa * (1.0 - cura_row), 0.0, 1.0)
        omb_row = 1.0 - blend_row
        R[ys, :] = jnp.where(pmask, R[ys, :] * omb_row + cr * blend_row,
                             R[ys, :])
        G[ys, :] = jnp.where(pmask, G[ys, :] * omb_row + cg * blend_row,
                             G[ys, :])
        B[ys, :] = jnp.where(pmask, B[ys, :] * omb_row + cb * blend_row,
                             B[ys, :])
        A[ys, :] = jnp.where(pmask,
                             jnp.clip(cura_row + blend_row, 0.0, 1.0),
                             cura_row)
        drow = D[ys, :]
        D[ys, :] = jnp.where(pmask & (blend_row > 0.01),
                             jnp.minimum(drow, dep), drow)
        return 0

    jax.lax.fori_loop(0, n_steps, step, 0)


def _blend(pid, pa2, dep2, r2, g2, b2, n_steps, interpret=False):
    fb = jax.ShapeDtypeStruct((_H * _W // _LANES, _LANES), jnp.float32)
    return pl.pallas_call(
        functools.partial(_blend_body, n_steps),
        in_specs=[pl.BlockSpec(memory_space=pltpu.VMEM)] * 6,
        out_specs=[pl.BlockSpec(memory_space=pltpu.VMEM)] * 5,
        out_shape=[fb] * 5,
        interpret=interpret,
    )(pid, pa2, dep2, r2, g2, b2)


def _rasterize_impl(positions, sizes, densities, colors,
                    camera_matrix, intrinsics, interpret=False):
    n = positions.shape[0]
    npad = ((n + _LANES - 1) // _LANES) * _LANES
    rows = npad // _LANES
    hom = jnp.concatenate([positions, jnp.ones((n, 1), positions.dtype)], axis=1)
    cam = hom @ camera_matrix.T
    cam3 = cam[:, :3] / cam[:, 3:4]
    scr = cam3 @ intrinsics.T
    xy = scr[:, :2] / scr[:, 2:3]
    dep = cam3[:, 2]
    ss = sizes * intrinsics[0, 0] / jnp.clip(dep, 0.1)

    def p2(v):
        return jnp.pad(v, (0, npad - n)).reshape(rows, _LANES)

    proj = _project(p2(xy[:, 0]), p2(xy[:, 1]), p2(dep), p2(ss),
                    p2(sizes), p2(densities), interpret=interpret)
    pa = proj[0].reshape(-1)[:n]
    pid = proj[1].reshape(-1)[:n]

    sort_w = jax.nn.softmax(dep / _TEMP)
    order = jnp.argsort(-sort_w)

    pad = npad - n
    pa_s = jnp.pad(pa[order], (0, pad)).reshape(rows, _LANES)
    dep_s = jnp.pad(dep[order], (0, pad)).reshape(rows, _LANES)
    pid_s = jnp.pad(pid[order], (0, pad)).astype(jnp.float32).reshape(rows, _LANES)
    col_s = colors[order]
    r_s = jnp.pad(col_s[:, 0], (0, pad)).reshape(rows, _LANES)
    g_s = jnp.pad(col_s[:, 1], (0, pad)).reshape(rows, _LANES)
    b_s = jnp.pad(col_s[:, 2], (0, pad)).reshape(rows, _LANES)

    pid_s, pa_s, dep_s, r_s, g_s, b_s = jax.lax.optimization_barrier(
        (pid_s, pa_s, dep_s, r_s, g_s, b_s))
    R, G, B, D, A = _blend(pid_s, pa_s, dep_s, r_s, g_s, b_s, n,
                           interpret=interpret)
    color = jnp.stack([R.reshape(_H, _W), G.reshape(_H, _W),
                       B.reshape(_H, _W)], axis=-1)
    return color, D.reshape(_H, _W), A.reshape(_H, _W)


def kernel(positions, sizes, densities, colors, camera_matrix, intrinsics):
    return _rasterize_impl(positions, sizes, densities, colors,
                           camera_matrix, intrinsics, interpret=False)
